# Initial kernel scaffold; baseline (speedup 1.0000x reference)
import jax
import jax.numpy as jnp
from jax.experimental import pallas as pl


def kernel(x_n0, x_n1, edge_index_e0, edge_index_e1, y_n0, w1e0_rel, b1e0, w1e0_root, w1e1_rel, b1e1, w1e1_root, w2e0_rel, b2e0, w2e0_root, w2e1_rel, b2e1, w2e1_root):
    raise NotImplementedError("write your pallas kernel here")


# speedup vs baseline: 1.6854x; 1.6854x over previous
import functools

import jax
import jax.numpy as jnp
from jax import lax
from jax.experimental import pallas as pl
from jax.experimental.pallas import tpu as pltpu
from jax.experimental.pallas import tpu_sc as plsc

N0 = 20000
N1 = 20000
D = 128
E = 320000
OUT = 128

NC = 2
NS = 16
HALF = N0 // NC
ACC_ROWS = HALF + 8
RCH = 80
NCH = HALF // RCH
EB = 128
NB = 160
SEG = 32
EPT = NB * EB
EPAD = NS * EPT


def _seg_sum_body(x_hbm, src_hbm, dst_hbm, out_hbm,
                  acc, src_v, off_v, rows_v, sem):
    c = lax.axis_index("c")
    s = lax.axis_index("s")
    base = c * HALF

    zero16 = jnp.zeros((16,), jnp.float32)

    @pl.loop(0, EB)
    def _(i):
        for j in range(8):
            rows_v[i, pl.ds(j * 16, 16)] = zero16

    @pl.loop(0, 8)
    def _(k):
        cid = s + k * NS

        @pl.when(cid < NCH)
        def _():
            pltpu.sync_copy(rows_v.at[pl.ds(0, RCH)],
                            acc.at[pl.ds(cid * RCH, RCH)])

    plsc.subcore_barrier()

    @pl.loop(0, NB // SEG)
    def _(seg):
        pltpu.sync_copy(src_hbm.at[s, pl.ds(seg * SEG, SEG)], src_v)
        pltpu.sync_copy(dst_hbm.at[s, pl.ds(seg * SEG, SEG)], off_v)

        @pl.loop(0, SEG)
        def _(i):
            for j in range(8):
                d = off_v[i, pl.ds(j * 16, 16)]
                off = d - base
                ok = (off >= 0) & (off < HALF)
                off_v[i, pl.ds(j * 16, 16)] = jnp.where(ok, off, HALF)

        @pl.loop(0, SEG)
        def _(g):
            pltpu.async_copy(x_hbm.at[src_v.at[g]], rows_v, sem).wait()
            pltpu.sync_copy(rows_v, acc.at[off_v.at[g]], add=True)

    plsc.subcore_barrier()

    @pl.loop(0, 8)
    def _(k):
        cid = s + k * NS

        @pl.when(cid < NCH)
        def _():
            pltpu.sync_copy(acc.at[pl.ds(cid * RCH, RCH)],
                            rows_v.at[pl.ds(0, RCH)])
            pltpu.sync_copy(rows_v.at[pl.ds(0, RCH)],
                            out_hbm.at[pl.ds(base + cid * RCH, RCH)])


def _seg_sum(x, src3, dst3, n_dst):
    mesh = plsc.VectorSubcoreMesh(core_axis_name="c", subcore_axis_name="s",
                                  num_cores=NC, num_subcores=NS)
    return pl.kernel(
        _seg_sum_body,
        out_type=jax.ShapeDtypeStruct((n_dst, D), jnp.float32),
        mesh=mesh,
        scratch_types=[
            pltpu.VMEM_SHARED((ACC_ROWS, D), jnp.float32),
            pltpu.VMEM((SEG, EB), jnp.int32),
            pltpu.VMEM((SEG, EB), jnp.int32),
            pltpu.VMEM((EB, D), jnp.float32),
            pltpu.SemaphoreType.DMA,
        ],
    )(x, src3, dst3)


BR = 2000
NBLK = N0 // BR


def _conv_relu_body(agg_ref, x_ref, wr_ref, b_ref, wo_ref, o_ref):
    y = (jnp.dot(agg_ref[...], wr_ref[...], preferred_element_type=jnp.float32)
         + jnp.dot(x_ref[...], wo_ref[...], preferred_element_type=jnp.float32)
         + b_ref[...])
    o_ref[...] = jnp.maximum(y, 0.0)


def _conv_relu(agg, x, w_rel, b, w_root):
    return pl.pallas_call(
        _conv_relu_body,
        grid=(NBLK,),
        in_specs=[
            pl.BlockSpec((BR, D), lambda i: (i, 0)),
            pl.BlockSpec((BR, D), lambda i: (i, 0)),
            pl.BlockSpec((D, OUT), lambda i: (0, 0)),
            pl.BlockSpec((1, OUT), lambda i: (0, 0)),
            pl.BlockSpec((D, OUT), lambda i: (0, 0)),
        ],
        out_specs=pl.BlockSpec((BR, OUT), lambda i: (i, 0)),
        out_shape=jax.ShapeDtypeStruct((N0, OUT), jnp.float32),
    )(agg, x, w_rel, b.reshape(1, OUT), w_root)


def _conv_loss_body(agg_ref, h_ref, wr_ref, b_ref, wo_ref, y_ref, o_ref):
    g = (jnp.dot(agg_ref[...], wr_ref[...], preferred_element_type=jnp.float32)
         + jnp.dot(h_ref[...], wo_ref[...], preferred_element_type=jnp.float32)
         + b_ref[...])
    m = jnp.max(g, axis=1, keepdims=True)
    lse = m[:, 0] + jnp.log(jnp.sum(jnp.exp(g - m), axis=1))
    yv = y_ref[0, 0, :]
    onehot = lax.broadcasted_iota(jnp.int32, (BR, OUT), 1) == yv[:, None]
    ylogit = jnp.sum(jnp.where(onehot, g, 0.0), axis=1)
    partial = jnp.sum(lse - ylogit)

    @pl.when(pl.program_id(0) == 0)
    def _():
        o_ref[...] = jnp.zeros((1, 1), jnp.float32)

    o_ref[...] += jnp.full((1, 1), partial * (1.0 / N0), jnp.float32)


def _conv_loss(agg, h, w_rel, b, w_root, y3):
    return pl.pallas_call(
        _conv_loss_body,
        grid=(NBLK,),
        in_specs=[
            pl.BlockSpec((BR, OUT), lambda i: (i, 0)),
            pl.BlockSpec((BR, OUT), lambda i: (i, 0)),
            pl.BlockSpec((OUT, OUT), lambda i: (0, 0)),
            pl.BlockSpec((1, OUT), lambda i: (0, 0)),
            pl.BlockSpec((OUT, OUT), lambda i: (0, 0)),
            pl.BlockSpec((1, 1, BR), lambda i: (i, 0, 0)),
        ],
        out_specs=pl.BlockSpec((1, 1), lambda i: (0, 0)),
        out_shape=jax.ShapeDtypeStruct((1, 1), jnp.float32),
    )(agg, h, w_rel, b.reshape(1, OUT), w_root, y3)


def _prep_edges(edge_index):
    src = edge_index[0].astype(jnp.int32)
    dst = edge_index[1].astype(jnp.int32)
    pad = EPAD - E
    src3 = jnp.concatenate(
        [src, jnp.zeros((pad,), jnp.int32)]).reshape(NS, NB, EB)
    dst3 = jnp.concatenate(
        [dst, jnp.full((pad,), 2 ** 28, jnp.int32)]).reshape(NS, NB, EB)
    return src3, dst3


def kernel(x_n0, x_n1, edge_index_e0, edge_index_e1, y_n0,
           w1e0_rel, b1e0, w1e0_root, w1e1_rel, b1e1, w1e1_root,
           w2e0_rel, b2e0, w2e0_root, w2e1_rel, b2e1, w2e1_root):
    src0, dst0 = _prep_edges(edge_index_e0)
    src1, dst1 = _prep_edges(edge_index_e1)

    agg_e0 = _seg_sum(x_n0, src0, dst0, N1)
    agg_e1 = _seg_sum(x_n1, src1, dst1, N0)
    h_n1 = _conv_relu(agg_e0, x_n1, w1e0_rel, b1e0, w1e0_root)
    h_n0 = _conv_relu(agg_e1, x_n0, w1e1_rel, b1e1, w1e1_root)

    agg2 = _seg_sum(h_n1, src1, dst1, N0)
    y3 = y_n0.astype(jnp.int32).reshape(NBLK, 1, BR)
    loss = _conv_loss(agg2, h_n0, w2e1_rel, b2e1, w2e1_root, y3)
    return loss[0, 0]


# speedup vs baseline: 1.7847x; 1.0589x over previous
import functools

import jax
import jax.numpy as jnp
from jax import lax
from jax.experimental import pallas as pl
from jax.experimental.pallas import tpu as pltpu
from jax.experimental.pallas import tpu_sc as plsc

N0 = 20000
N1 = 20000
D = 128
E = 320000
OUT = 128

NC = 2
NS = 16
HALF = N0 // NC
ACC_ROWS = HALF + 8
RCH = 80
NCH = HALF // RCH
EB = 128
NB = 160
SEG = 32
EPT = NB * EB
EPAD = NS * EPT


def _seg_sum_body(x_hbm, src_hbm, dst_hbm, out_hbm,
                  acc, src_v, off_v, buf0, buf1, g0, g1, s0, s1):
    c = lax.axis_index("c")
    s = lax.axis_index("s")
    base = c * HALF

    zero16 = jnp.zeros((16,), jnp.float32)

    @pl.loop(0, EB)
    def _(i):
        for j in range(8):
            buf0[i, pl.ds(j * 16, 16)] = zero16

    @pl.loop(0, 8)
    def _(k):
        cid = s + k * NS

        @pl.when(cid < NCH)
        def _():
            pltpu.sync_copy(buf0.at[pl.ds(0, RCH)],
                            acc.at[pl.ds(cid * RCH, RCH)])

    plsc.subcore_barrier()

    @pl.loop(0, NB // SEG)
    def _(seg):
        pltpu.sync_copy(src_hbm.at[s, pl.ds(seg * SEG, SEG)], src_v)
        pltpu.sync_copy(dst_hbm.at[s, pl.ds(seg * SEG, SEG)], off_v)

        @pl.loop(0, SEG)
        def _(i):
            for j in range(8):
                d = off_v[i, pl.ds(j * 16, 16)]
                off = d - base
                ok = (off >= 0) & (off < HALF)
                off_v[i, pl.ds(j * 16, 16)] = jnp.where(ok, off, HALF)

        pltpu.async_copy(x_hbm.at[src_v.at[0]], buf0, g0)

        @pl.loop(0, SEG // 2)
        def _(h):
            b0 = 2 * h
            b1 = b0 + 1
            pltpu.make_async_copy(x_hbm.at[src_v.at[b0]], buf0, g0).wait()

            @pl.when(h >= 1)
            def _():
                pltpu.make_async_copy(buf1, acc.at[off_v.at[b1 - 2]],
                                      s1).wait()

            pltpu.async_copy(x_hbm.at[src_v.at[b1]], buf1, g1)
            pltpu.async_copy(buf0, acc.at[off_v.at[b0]], s0, add=True)
            pltpu.make_async_copy(x_hbm.at[src_v.at[b1]], buf1, g1).wait()
            pltpu.make_async_copy(buf0, acc.at[off_v.at[b0]], s0).wait()

            @pl.when(h + 1 < SEG // 2)
            def _():
                pltpu.async_copy(x_hbm.at[src_v.at[b0 + 2]], buf0, g0)

            pltpu.async_copy(buf1, acc.at[off_v.at[b1]], s1, add=True)

        pltpu.make_async_copy(buf1, acc.at[off_v.at[SEG - 1]], s1).wait()

    plsc.subcore_barrier()

    @pl.loop(0, 8)
    def _(k):
        cid = s + k * NS

        @pl.when(cid < NCH)
        def _():
            pltpu.sync_copy(acc.at[pl.ds(cid * RCH, RCH)],
                            buf0.at[pl.ds(0, RCH)])
            pltpu.sync_copy(buf0.at[pl.ds(0, RCH)],
                            out_hbm.at[pl.ds(base + cid * RCH, RCH)])


def _seg_sum(x, src3, dst3, n_dst):
    mesh = plsc.VectorSubcoreMesh(core_axis_name="c", subcore_axis_name="s",
                                  num_cores=NC, num_subcores=NS)
    return pl.kernel(
        _seg_sum_body,
        out_type=jax.ShapeDtypeStruct((n_dst, D), jnp.float32),
        mesh=mesh,
        scratch_types=[
            pltpu.VMEM_SHARED((ACC_ROWS, D), jnp.float32),
            pltpu.VMEM((SEG, EB), jnp.int32),
            pltpu.VMEM((SEG, EB), jnp.int32),
            pltpu.VMEM((EB, D), jnp.float32),
            pltpu.VMEM((EB, D), jnp.float32),
            pltpu.SemaphoreType.DMA,
            pltpu.SemaphoreType.DMA,
            pltpu.SemaphoreType.DMA,
            pltpu.SemaphoreType.DMA,
        ],
    )(x, src3, dst3)


BR = 2000
NBLK = N0 // BR


def _conv_relu_body(agg_ref, x_ref, wr_ref, b_ref, wo_ref, o_ref):
    y = (jnp.dot(agg_ref[...], wr_ref[...], preferred_element_type=jnp.float32)
         + jnp.dot(x_ref[...], wo_ref[...], preferred_element_type=jnp.float32)
         + b_ref[...])
    o_ref[...] = jnp.maximum(y, 0.0)


def _conv_relu(agg, x, w_rel, b, w_root):
    return pl.pallas_call(
        _conv_relu_body,
        grid=(NBLK,),
        in_specs=[
            pl.BlockSpec((BR, D), lambda i: (i, 0)),
            pl.BlockSpec((BR, D), lambda i: (i, 0)),
            pl.BlockSpec((D, OUT), lambda i: (0, 0)),
            pl.BlockSpec((1, OUT), lambda i: (0, 0)),
            pl.BlockSpec((D, OUT), lambda i: (0, 0)),
        ],
        out_specs=pl.BlockSpec((BR, OUT), lambda i: (i, 0)),
        out_shape=jax.ShapeDtypeStruct((N0, OUT), jnp.float32),
    )(agg, x, w_rel, b.reshape(1, OUT), w_root)


def _conv_loss_body(agg_ref, h_ref, wr_ref, b_ref, wo_ref, y_ref, o_ref):
    g = (jnp.dot(agg_ref[...], wr_ref[...], preferred_element_type=jnp.float32)
         + jnp.dot(h_ref[...], wo_ref[...], preferred_element_type=jnp.float32)
         + b_ref[...])
    m = jnp.max(g, axis=1, keepdims=True)
    lse = m[:, 0] + jnp.log(jnp.sum(jnp.exp(g - m), axis=1))
    yv = y_ref[0, 0, :]
    onehot = lax.broadcasted_iota(jnp.int32, (BR, OUT), 1) == yv[:, None]
    ylogit = jnp.sum(jnp.where(onehot, g, 0.0), axis=1)
    partial = jnp.sum(lse - ylogit)

    @pl.when(pl.program_id(0) == 0)
    def _():
        o_ref[...] = jnp.zeros((1, 1), jnp.float32)

    o_ref[...] += jnp.full((1, 1), partial * (1.0 / N0), jnp.float32)


def _conv_loss(agg, h, w_rel, b, w_root, y3):
    return pl.pallas_call(
        _conv_loss_body,
        grid=(NBLK,),
        in_specs=[
            pl.BlockSpec((BR, OUT), lambda i: (i, 0)),
            pl.BlockSpec((BR, OUT), lambda i: (i, 0)),
            pl.BlockSpec((OUT, OUT), lambda i: (0, 0)),
            pl.BlockSpec((1, OUT), lambda i: (0, 0)),
            pl.BlockSpec((OUT, OUT), lambda i: (0, 0)),
            pl.BlockSpec((1, 1, BR), lambda i: (i, 0, 0)),
        ],
        out_specs=pl.BlockSpec((1, 1), lambda i: (0, 0)),
        out_shape=jax.ShapeDtypeStruct((1, 1), jnp.float32),
    )(agg, h, w_rel, b.reshape(1, OUT), w_root, y3)


def _prep_edges(edge_index):
    src = edge_index[0].astype(jnp.int32)
    dst = edge_index[1].astype(jnp.int32)
    pad = EPAD - E
    src3 = jnp.concatenate(
        [src, jnp.zeros((pad,), jnp.int32)]).reshape(NS, NB, EB)
    dst3 = jnp.concatenate(
        [dst, jnp.full((pad,), 2 ** 28, jnp.int32)]).reshape(NS, NB, EB)
    return src3, dst3


def kernel(x_n0, x_n1, edge_index_e0, edge_index_e1, y_n0,
           w1e0_rel, b1e0, w1e0_root, w1e1_rel, b1e1, w1e1_root,
           w2e0_rel, b2e0, w2e0_root, w2e1_rel, b2e1, w2e1_root):
    src0, dst0 = _prep_edges(edge_index_e0)
    src1, dst1 = _prep_edges(edge_index_e1)

    agg_e0 = _seg_sum(x_n0, src0, dst0, N1)
    agg_e1 = _seg_sum(x_n1, src1, dst1, N0)
    h_n1 = _conv_relu(agg_e0, x_n1, w1e0_rel, b1e0, w1e0_root)
    h_n0 = _conv_relu(agg_e1, x_n0, w1e1_rel, b1e1, w1e1_root)

    agg2 = _seg_sum(h_n1, src1, dst1, N0)
    y3 = y_n0.astype(jnp.int32).reshape(NBLK, 1, BR)
    loss = _conv_loss(agg2, h_n0, w2e1_rel, b2e1, w2e1_root, y3)
    return loss[0, 0]
